# baseline (speedup 1.0000x reference)
import functools

import jax
import jax.numpy as jnp
from jax import lax
from jax.experimental import pallas as pl
from jax.experimental.pallas import tpu as pltpu
from jax.experimental.pallas import tpu_sc as plsc

D = 64
DP = 128
NFIELD = 2
ROLES = 7
NDOT = 6
NC = 2
NS = 16
L = 16
NW = NC * NS
B = 16384
VOCAB = 100000
BW = B // NW
CB = 32
NCHUNK = BW // CB
RPC = CB * ROLES
IDX_W = 112
NSUB = RPC // IDX_W

_mesh = plsc.VectorSubcoreMesh(
    core_axis_name="c", subcore_axis_name="s", num_cores=NC, num_subcores=NS)


@functools.partial(
    pl.kernel,
    out_type=jax.ShapeDtypeStruct((NW, NDOT, BW), jnp.float32),
    mesh=_mesh,
    compiler_params=pltpu.CompilerParams(
        needs_layout_passes=False, use_tc_tiling_on_sc=True),
    scratch_types=[
        pltpu.VMEM((BW * ROLES,), jnp.int32),
        pltpu.VMEM((BW * ROLES,), jnp.int32),
        pltpu.VMEM((2, RPC, DP), jnp.float32),
        pltpu.VMEM((2, RPC, DP), jnp.float32),
        pltpu.VMEM((NDOT, 4 * CB), jnp.float32),
        pltpu.SemaphoreType.DMA,
        pltpu.SemaphoreType.DMA,
    ],
)
def _sc_scores(idx0_hbm, idx1_hbm, t0_hbm, t1_hbm, out_hbm,
               idx0_v, idx1_v, buf0, buf1, obuf, sem0, sem1):
    cid = lax.axis_index("c")
    sid = lax.axis_index("s")
    wid = sid * NC + cid
    zero_i = jnp.zeros((L,), jnp.int32)
    one_i = jnp.ones((L,), jnp.int32)
    lane = lax.iota(jnp.int32, L)
    pltpu.sync_copy(idx0_hbm.at[pl.ds(wid * BW * ROLES, BW * ROLES)], idx0_v)
    pltpu.sync_copy(idx1_hbm.at[pl.ds(wid * BW * ROLES, BW * ROLES)], idx1_v)
    sems = (sem0, sem1)

    def issue(c, sl):
        handles = []
        for off, n in ((0, 128), (128, RPC - 128)):
            rows = pl.ds(off, n)
            isl0 = idx0_v.at[pl.ds(c * RPC + off, n)]
            isl1 = idx1_v.at[pl.ds(c * RPC + off, n)]
            handles.append(pltpu.async_copy(
                t0_hbm.at[isl0], buf0.at[sl, rows, :], sems[sl]))
            handles.append(pltpu.async_copy(
                t1_hbm.at[isl1], buf1.at[sl, rows, :], sems[sl]))
        return handles

    pending = issue(0, 0)
    for c in range(NCHUNK):
        sl = c & 1
        for h in pending:
            h.wait()
        if c + 1 < NCHUNK:
            pending = issue(c + 1, 1 - sl)
        cbuf0, cbuf1 = buf0.at[sl], buf1.at[sl]
        for g in range(CB // L):
            slots = [(lane + g * L) * ROLES + r for r in range(ROLES)]
            p0, p1 = [], []
            for r in range(ROLES):
                w0 = plsc.load_gather(cbuf0, [slots[r], zero_i + D])
                w1 = plsc.load_gather(cbuf0, [slots[r], zero_i + (D + 1)])
                a = 1.0 / (1.0 + jnp.exp(w1 - w0))
                p0.append(a)
                p1.append(1.0 - a)

            def body(d, accs, slots=slots, p0=p0, p1=p1, cbuf0=cbuf0, cbuf1=cbuf1):
                dv = (d & ~15) + (((d & 15) + lane) & 15)
                e = []
                for r in range(ROLES):
                    g0 = plsc.load_gather(cbuf0, [slots[r], dv])
                    g1 = plsc.load_gather(cbuf1, [slots[r], dv])
                    e.append(p0[r] * g0 + p1[r] * g1)
                out = [accs[0] + e[0] * e[1]]
                for i in range(NDOT - 1):
                    out.append(accs[1 + i] + e[0] * e[2 + i])
                return tuple(out)

            acc = lax.fori_loop(0, D, body, (jnp.zeros((L,), jnp.float32),) * NDOT)
            for k in range(NDOT):
                obuf[k, pl.ds((c % 4) * CB + g * L, L)] = acc[k]
        if c % 4 == 3:
            pltpu.sync_copy(obuf, out_hbm.at[wid, :, pl.ds((c // 4) * 4 * CB, 4 * CB)])


def _loss_body(s_ref, o_ref):
    x = jnp.clip(s_ref[...], -10.0, 10.0)
    seg = (lax.broadcasted_iota(jnp.int32, x.shape, 0) * 2
           + lax.broadcasted_iota(jnp.int32, x.shape, 1) // BW)
    z = jnp.where(seg % NDOT == 0, -x, x)
    o_ref[0, 0] = jnp.sum(jnp.logaddexp(z, 0.0)) * (1.0 / B)


_loss = pl.pallas_call(
    _loss_body,
    out_shape=jax.ShapeDtypeStruct((1, 1), jnp.float32),
    out_specs=pl.BlockSpec(memory_space=pltpu.SMEM),
)


def kernel(centers, contexts, neg_contexts, W_center0, W_center1, W_weights):
    centers = centers.astype(jnp.int32)
    contexts = contexts.astype(jnp.int32)
    neg_contexts = neg_contexts.astype(jnp.int32)
    idx0 = jnp.concatenate(
        [centers[:, 0:1], contexts[:, 0:1], neg_contexts[:, 0::2]], axis=1)
    idx1 = jnp.concatenate(
        [centers[:, 1:2], contexts[:, 1:2], neg_contexts[:, 1::2]], axis=1)
    idx0 = idx0.reshape(-1)
    idx1 = idx1.reshape(-1)
    t0p = jax.lax.dynamic_update_slice(
        jnp.pad(W_center0, ((0, 0), (0, DP - D))), W_weights, (0, D))
    t1p = jnp.pad(W_center1, ((0, 0), (0, DP - D)))
    scores = _sc_scores(idx0, idx1, t0p, t1p)
    return _loss(scores.reshape(NW * NDOT // 2, BW * 2))[0, 0]


# speedup vs baseline: 2.1451x; 2.1451x over previous
import functools

import jax
import jax.numpy as jnp
from jax import lax
from jax.experimental import pallas as pl
from jax.experimental.pallas import tpu as pltpu
from jax.experimental.pallas import tpu_sc as plsc

D = 64
DP = 128
NFIELD = 2
ROLES = 7
NDOT = 6
NC = 2
NS = 16
L = 16
NW = NC * NS
B = 16384
VOCAB = 100000
BW = B // NW
CB = 32
NCHUNK = BW // CB
RPC = CB * ROLES
IDX_W = 112
NSUB = RPC // IDX_W

_mesh = plsc.VectorSubcoreMesh(
    core_axis_name="c", subcore_axis_name="s", num_cores=NC, num_subcores=NS)


@functools.partial(
    pl.kernel,
    out_type=jax.ShapeDtypeStruct((NW, NDOT, BW), jnp.float32),
    mesh=_mesh,
    compiler_params=pltpu.CompilerParams(
        needs_layout_passes=False, use_tc_tiling_on_sc=True),
    scratch_types=[
        pltpu.VMEM((CB * 2 * ROLES,), jnp.int32),
        pltpu.VMEM((RPC,), jnp.int32),
        pltpu.VMEM((RPC,), jnp.int32),
        pltpu.VMEM((2, RPC, DP), jnp.float32),
        pltpu.VMEM((2, RPC, DP), jnp.float32),
        pltpu.VMEM((NDOT, 4 * CB), jnp.float32),
        pltpu.SemaphoreType.DMA,
        pltpu.SemaphoreType.DMA,
    ],
)
def _sc_scores(ia_hbm, t0_hbm, t1_hbm, out_hbm,
               stage, idx0_v, idx1_v, buf0, buf1, obuf, sem0, sem1):
    cid = lax.axis_index("c")
    sid = lax.axis_index("s")
    wid = sid * NC + cid
    zero_i = jnp.zeros((L,), jnp.int32)
    one_i = jnp.ones((L,), jnp.int32)
    lane = lax.iota(jnp.int32, L)
    sems = (sem0, sem1)

    def issue(c, sl):
        pltpu.sync_copy(
            ia_hbm.at[pl.ds((wid * BW + c * CB) * 2 * ROLES, CB * 2 * ROLES)],
            stage)
        for gb in range(RPC // L):
            sv = (lane + gb * L) * 2
            idx0_v[pl.ds(gb * L, L)] = plsc.load_gather(stage, [sv])
            idx1_v[pl.ds(gb * L, L)] = plsc.load_gather(stage, [sv + 1])
        handles = []
        for off, n in ((0, 128), (128, RPC - 128)):
            rows = pl.ds(off, n)
            handles.append(pltpu.async_copy(
                t0_hbm.at[idx0_v.at[pl.ds(off, n)]], buf0.at[sl, rows, :], sems[sl]))
            handles.append(pltpu.async_copy(
                t1_hbm.at[idx1_v.at[pl.ds(off, n)]], buf1.at[sl, rows, :], sems[sl]))
        return handles

    pending = issue(0, 0)
    for c in range(NCHUNK):
        sl = c & 1
        for h in pending:
            h.wait()
        if c + 1 < NCHUNK:
            pending = issue(c + 1, 1 - sl)
        cbuf0, cbuf1 = buf0.at[sl], buf1.at[sl]
        for g in range(CB // L):
            slots = [(lane + g * L) * ROLES + r for r in range(ROLES)]
            p0, p1 = [], []
            for r in range(ROLES):
                w0 = plsc.load_gather(cbuf0, [slots[r], zero_i + D])
                w1 = plsc.load_gather(cbuf0, [slots[r], zero_i + (D + 1)])
                a = 1.0 / (1.0 + jnp.exp(w1 - w0))
                p0.append(a)
                p1.append(1.0 - a)

            def body(d, accs, slots=slots, p0=p0, p1=p1, cbuf0=cbuf0, cbuf1=cbuf1):
                dv = (d & ~15) + (((d & 15) + lane) & 15)
                e = []
                for r in range(ROLES):
                    g0 = plsc.load_gather(cbuf0, [slots[r], dv])
                    g1 = plsc.load_gather(cbuf1, [slots[r], dv])
                    e.append(p0[r] * g0 + p1[r] * g1)
                out = [accs[0] + e[0] * e[1]]
                for i in range(NDOT - 1):
                    out.append(accs[1 + i] + e[0] * e[2 + i])
                return tuple(out)

            acc = lax.fori_loop(0, D, body, (jnp.zeros((L,), jnp.float32),) * NDOT)
            for k in range(NDOT):
                obuf[k, pl.ds((c % 4) * CB + g * L, L)] = acc[k]
        if c % 4 == 3:
            pltpu.sync_copy(obuf, out_hbm.at[wid, :, pl.ds((c // 4) * 4 * CB, 4 * CB)])


def _loss_body(s_ref, o_ref):
    x = jnp.clip(s_ref[...], -10.0, 10.0)
    seg = (lax.broadcasted_iota(jnp.int32, x.shape, 0) * 2
           + lax.broadcasted_iota(jnp.int32, x.shape, 1) // BW)
    z = jnp.where(seg % NDOT == 0, -x, x)
    o_ref[0, 0] = jnp.sum(jnp.logaddexp(z, 0.0)) * (1.0 / B)


_loss = pl.pallas_call(
    _loss_body,
    out_shape=jax.ShapeDtypeStruct((1, 1), jnp.float32),
    out_specs=pl.BlockSpec(memory_space=pltpu.SMEM),
)


def kernel(centers, contexts, neg_contexts, W_center0, W_center1, W_weights):
    centers = centers.astype(jnp.int32)
    contexts = contexts.astype(jnp.int32)
    neg_contexts = neg_contexts.astype(jnp.int32)
    idx_all = jnp.concatenate(
        [centers, contexts, neg_contexts], axis=1).reshape(-1)
    t0p = jnp.concatenate(
        [W_center0, W_weights,
         jnp.zeros((VOCAB, DP - D - NFIELD), jnp.float32)], axis=1)
    t1p = jnp.concatenate(
        [W_center1, jnp.zeros((VOCAB, DP - D), jnp.float32)], axis=1)
    scores = _sc_scores(idx_all, t0p, t1p)
    return _loss(scores.reshape(NW * NDOT // 2, BW * 2))[0, 0]


# speedup vs baseline: 2.1600x; 1.0070x over previous
import functools

import jax
import jax.numpy as jnp
from jax import lax
from jax.experimental import pallas as pl
from jax.experimental.pallas import tpu as pltpu
from jax.experimental.pallas import tpu_sc as plsc

D = 64
DP = 128
NFIELD = 2
ROLES = 7
NDOT = 6
NC = 2
NS = 16
L = 16
NW = NC * NS
B = 16384
VOCAB = 100000
BW = B // NW
CB = 32
NCHUNK = BW // CB
RPC = CB * ROLES
IDX_W = 112
NSUB = RPC // IDX_W

_mesh = plsc.VectorSubcoreMesh(
    core_axis_name="c", subcore_axis_name="s", num_cores=NC, num_subcores=NS)


@functools.partial(
    pl.kernel,
    out_type=jax.ShapeDtypeStruct((NW, NDOT, BW), jnp.float32),
    mesh=_mesh,
    compiler_params=pltpu.CompilerParams(
        needs_layout_passes=False, use_tc_tiling_on_sc=True),
    scratch_types=[
        pltpu.VMEM((BW * ROLES,), jnp.int32),
        pltpu.VMEM((BW * ROLES,), jnp.int32),
        pltpu.VMEM((2, RPC, DP), jnp.float32),
        pltpu.VMEM((2, RPC, DP), jnp.float32),
        pltpu.VMEM((NDOT, 4 * CB), jnp.float32),
        pltpu.SemaphoreType.DMA,
        pltpu.SemaphoreType.DMA,
    ],
)
def _sc_scores(idx0_hbm, idx1_hbm, t0_hbm, t1_hbm, out_hbm,
               idx0_v, idx1_v, buf0, buf1, obuf, sem0, sem1):
    cid = lax.axis_index("c")
    sid = lax.axis_index("s")
    wid = sid * NC + cid
    zero_i = jnp.zeros((L,), jnp.int32)
    one_i = jnp.ones((L,), jnp.int32)
    lane = lax.iota(jnp.int32, L)
    pltpu.sync_copy(idx0_hbm.at[pl.ds(wid * BW * ROLES, BW * ROLES)], idx0_v)
    pltpu.sync_copy(idx1_hbm.at[pl.ds(wid * BW * ROLES, BW * ROLES)], idx1_v)
    sems = (sem0, sem1)

    def issue(c, sl):
        handles = []
        for off, n in ((0, 128), (128, RPC - 128)):
            rows = pl.ds(off, n)
            isl0 = idx0_v.at[pl.ds(c * RPC + off, n)]
            isl1 = idx1_v.at[pl.ds(c * RPC + off, n)]
            handles.append(pltpu.async_copy(
                t0_hbm.at[isl0], buf0.at[sl, rows, :], sems[sl]))
            handles.append(pltpu.async_copy(
                t1_hbm.at[isl1], buf1.at[sl, rows, :], sems[sl]))
        return handles

    pending = issue(0, 0)
    for c in range(NCHUNK):
        sl = c & 1
        for h in pending:
            h.wait()
        if c + 1 < NCHUNK:
            pending = issue(c + 1, 1 - sl)
        cbuf0, cbuf1 = buf0.at[sl], buf1.at[sl]
        for g in range(CB // L):
            slots = [(lane + g * L) * ROLES + r for r in range(ROLES)]
            p0, p1 = [], []
            for r in range(ROLES):
                w0 = plsc.load_gather(cbuf0, [slots[r], zero_i + D])
                w1 = plsc.load_gather(cbuf0, [slots[r], zero_i + (D + 1)])
                a = 1.0 / (1.0 + jnp.exp(w1 - w0))
                p0.append(a)
                p1.append(1.0 - a)

            def body(d, accs, slots=slots, p0=p0, p1=p1, cbuf0=cbuf0, cbuf1=cbuf1):
                dv = (d & ~15) + (((d & 15) + lane) & 15)
                e = []
                for r in range(ROLES):
                    g0 = plsc.load_gather(cbuf0, [slots[r], dv])
                    g1 = plsc.load_gather(cbuf1, [slots[r], dv])
                    e.append(p0[r] * g0 + p1[r] * g1)
                out = [accs[0] + e[0] * e[1]]
                for i in range(NDOT - 1):
                    out.append(accs[1 + i] + e[0] * e[2 + i])
                return tuple(out)

            acc = lax.fori_loop(0, D, body, (jnp.zeros((L,), jnp.float32),) * NDOT)
            for k in range(NDOT):
                obuf[k, pl.ds((c % 4) * CB + g * L, L)] = acc[k]
        if c % 4 == 3:
            pltpu.sync_copy(obuf, out_hbm.at[wid, :, pl.ds((c // 4) * 4 * CB, 4 * CB)])


def _loss_body(s_ref, o_ref):
    x = jnp.clip(s_ref[...], -10.0, 10.0)
    seg = (lax.broadcasted_iota(jnp.int32, x.shape, 0) * 2
           + lax.broadcasted_iota(jnp.int32, x.shape, 1) // BW)
    z = jnp.where(seg % NDOT == 0, -x, x)
    o_ref[0, 0] = jnp.sum(jnp.logaddexp(z, 0.0)) * (1.0 / B)


_loss = pl.pallas_call(
    _loss_body,
    out_shape=jax.ShapeDtypeStruct((1, 1), jnp.float32),
    out_specs=pl.BlockSpec(memory_space=pltpu.SMEM),
)


def kernel(centers, contexts, neg_contexts, W_center0, W_center1, W_weights):
    centers = centers.astype(jnp.int32)
    contexts = contexts.astype(jnp.int32)
    neg_contexts = neg_contexts.astype(jnp.int32)
    idx0 = jnp.concatenate(
        [centers[:, 0:1], contexts[:, 0:1], neg_contexts[:, 0::2]], axis=1)
    idx1 = jnp.concatenate(
        [centers[:, 1:2], contexts[:, 1:2], neg_contexts[:, 1::2]], axis=1)
    idx0 = idx0.reshape(-1)
    idx1 = idx1.reshape(-1)
    t0p = jnp.concatenate(
        [W_center0, W_weights,
         jnp.zeros((VOCAB, DP - D - NFIELD), jnp.float32)], axis=1)
    t1p = jnp.concatenate(
        [W_center1, jnp.zeros((VOCAB, DP - D), jnp.float32)], axis=1)
    scores = _sc_scores(idx0, idx1, t0p, t1p)
    return _loss(scores.reshape(NW * NDOT // 2, BW * 2))[0, 0]
